# baseline (speedup 1.0000x reference)
import functools

import jax
import jax.numpy as jnp
from jax import lax
from jax.experimental import pallas as pl
from jax.experimental.pallas import tpu as pltpu
from jax.experimental.pallas import tpu_sc as plsc

N_NODES = 10000
NPAD = 10112
E_EDGES = 320000
EB = 64
NW = 32
NB = 168
EPAD = NW * NB * EB
RPS = NPAD // 16
ROW_CHUNKS = [(i * EB, EB) for i in range(RPS // EB)] + [((RPS // EB) * EB, RPS % EB)]
RBLK = 1000
GRID = N_NODES // RBLK


def _ln(x, g, b, eps=1e-5):
    m = jnp.mean(x, axis=-1, keepdims=True)
    v = jnp.mean((x - m) ** 2, axis=-1, keepdims=True)
    return (x - m) * lax.rsqrt(v + eps) * g + b


def _elu(x):
    return jnp.where(x > 0, x, jnp.exp(x) - 1.0)


def _dinv(deg0_ref, deg1_ref):
    d = deg0_ref[:, 0:1] + deg1_ref[:, 0:1] + 2.0
    return lax.rsqrt(d)


def _enc_body(om_ref, tp_ref, wo1, bo1, go1, beo1, wo2, bo2, go2, beo2,
              wt, bt, gt, bet, wg1, h0_ref, xw1_ref):
    h = jnp.dot(om_ref[...], wo1[...], preferred_element_type=jnp.float32) + bo1[...]
    h = jax.nn.relu(_ln(h, go1[...], beo1[...]))
    h = jnp.dot(h, wo2[...], preferred_element_type=jnp.float32) + bo2[...]
    h_om = _ln(h, go2[...], beo2[...])
    ht = jnp.dot(tp_ref[...], wt[...], preferred_element_type=jnp.float32) + bt[...]
    h_tp = jax.nn.relu(_ln(ht, gt[...], bet[...]))
    h0 = jnp.concatenate([h_om, h_tp], axis=-1)
    h0_ref[...] = h0
    xw1_ref[...] = jnp.dot(h0, wg1[...], preferred_element_type=jnp.float32)


def _scale_body(deg0_ref, deg1_ref, xw_ref, a_ref):
    a_ref[...] = _dinv(deg0_ref, deg1_ref) * xw_ref[...]


def _layer_body(s0_ref, s1_ref, xw_ref, h0_ref, deg0_ref, deg1_ref,
                bg, gl, bel, wg2, xw2_ref, a2_ref):
    dinv = _dinv(deg0_ref, deg1_ref)
    c = dinv * (s0_ref[...] + s1_ref[...]) + (2.0 * dinv * dinv) * xw_ref[...] + bg[...]
    h = 0.8 * c + 0.2 * h0_ref[...]
    h = _elu(_ln(h, gl[...], bel[...]))
    xw2 = jnp.dot(h, wg2[...], preferred_element_type=jnp.float32)
    xw2_ref[...] = xw2
    a2_ref[...] = dinv * xw2


def _final_body(s0_ref, s1_ref, xw_ref, h0_ref, deg0_ref, deg1_ref,
                bg, gl, bel, wc1, bc1, gc, bec, wc2r, bc2, out_ref):
    dinv = _dinv(deg0_ref, deg1_ref)
    c = dinv * (s0_ref[...] + s1_ref[...]) + (2.0 * dinv * dinv) * xw_ref[...] + bg[...]
    h = 0.8 * c + 0.2 * h0_ref[...]
    h = _elu(_ln(h, gl[...], bel[...]))
    z = jnp.dot(h, wc1[...], preferred_element_type=jnp.float32) + bc1[...]
    z = jax.nn.relu(_ln(z, gc[...], bec[...]))
    out_ref[...] = jnp.sum(z * wc2r[...], axis=-1, keepdims=True) + bc2[...]


def _row_spec(cols):
    return pl.BlockSpec((RBLK, cols), lambda i: (i, 0))


def _full_spec(shape):
    nd = len(shape)
    return pl.BlockSpec(shape, lambda i: (0,) * nd)


def _tc_call(body, ins, full_ins, outs):
    in_specs = ([_row_spec(x.shape[1]) for x in ins]
                + [_full_spec(w.shape) for w in full_ins])
    out_specs = [_row_spec(c) for c in outs]
    out_shape = [jax.ShapeDtypeStruct((N_NODES, c), jnp.float32) for c in outs]
    res = pl.pallas_call(
        body, grid=(GRID,), in_specs=in_specs,
        out_specs=out_specs if len(outs) > 1 else out_specs[0],
        out_shape=out_shape if len(outs) > 1 else out_shape[0],
    )(*ins, *full_ins)
    return res


def _zero_rows(buf, ncols):
    nchunks = ncols // 16

    def zrow(i, c):
        for j in range(nchunks):
            buf[i, pl.ds(j * 16, 16)] = jnp.zeros((16,), jnp.float32)
        return c

    lax.fori_loop(0, buf.shape[0], zrow, 0)


def _fill_ones(buf, ncols):
    nchunks = ncols // 16

    def orow(i, c):
        for j in range(nchunks):
            buf[i, pl.ds(j * 16, 16)] = jnp.ones((16,), jnp.float32)
        return c

    lax.fori_loop(0, buf.shape[0], orow, 0)


CH = 24
NCHUNK = NB // CH
RING = 3


def _scatter_rows_body(a_hbm, src2d, dst2d, out_hbm,
                       src_v, dst_v, b0, b1, b2, acc, g0, g1, g2):
    cid = lax.axis_index("c")
    sid = lax.axis_index("s")
    wid = sid * 2 + cid
    bufs = [b0, b1, b2]
    sems = [g0, g1, g2]
    _zero_rows(b0, 128)
    base = sid * RPS
    for off, sz in ROW_CHUNKS:
        pltpu.sync_copy(b0.at[pl.ds(0, sz)], acc.at[pl.ds(base + off, sz)])
    plsc.subcore_barrier()

    def fire(t, bi):
        pltpu.async_copy(a_hbm.at[src_v.at[t]], bufs[bi], sems[bi])

    def drain(bi):
        pltpu.make_async_copy(a_hbm.at[pl.ds(0, EB)], bufs[bi], sems[bi]).wait()

    def scat(t, bi):
        pltpu.sync_copy(bufs[bi], acc.at[dst_v.at[t]], add=True)

    def chunk(c, carry):
        pltpu.sync_copy(src2d.at[pl.ds(wid * NB + c * CH, CH)], src_v)
        pltpu.sync_copy(dst2d.at[pl.ds(wid * NB + c * CH, CH)], dst_v)
        fire(0, 0)
        fire(1, 1)

        def trip(k, cc):
            for j in range(RING):
                t = RING * k + j
                drain(j)

                @pl.when(t + 2 < CH)
                def _():
                    fire(t + 2, (j + 2) % RING)

                scat(t, j)
            return cc

        lax.fori_loop(0, CH // RING, trip, 0)
        return carry

    lax.fori_loop(0, NCHUNK, chunk, 0)
    plsc.subcore_barrier()
    for off, sz in ROW_CHUNKS:
        pltpu.sync_copy(acc.at[pl.ds(base + off, sz)], b0.at[pl.ds(0, sz)])
        pltpu.sync_copy(b0.at[pl.ds(0, sz)], out_hbm.at[cid, pl.ds(base + off, sz)])


def _degree_body(dst2d, out_hbm, dst_v, ones_v, zbuf, acc, _unused_sem):
    cid = lax.axis_index("c")
    sid = lax.axis_index("s")
    wid = sid * 2 + cid
    pltpu.sync_copy(dst2d.at[pl.ds(wid * NB, NB)], dst_v)
    _zero_rows(zbuf, 128)
    _fill_ones(ones_v, 128)
    base = sid * RPS
    for off, sz in ROW_CHUNKS:
        pltpu.sync_copy(zbuf.at[pl.ds(0, sz)], acc.at[pl.ds(base + off, sz)])
    plsc.subcore_barrier()

    def body(t, c):
        pltpu.sync_copy(ones_v, acc.at[dst_v.at[t]], add=True)
        return c

    lax.fori_loop(0, NB, body, 0)
    plsc.subcore_barrier()
    for off, sz in ROW_CHUNKS:
        pltpu.sync_copy(acc.at[pl.ds(base + off, sz)], zbuf.at[pl.ds(0, sz)])
        pltpu.sync_copy(zbuf.at[pl.ds(0, sz)], out_hbm.at[cid, pl.ds(base + off, sz)])


def _sc_mesh():
    return plsc.VectorSubcoreMesh(core_axis_name="c", subcore_axis_name="s",
                                  num_cores=2, num_subcores=16)


def _sc_scatter(a, src2d, dst2d):
    k = pl.kernel(
        _scatter_rows_body,
        out_type=jax.ShapeDtypeStruct((2, NPAD, 128), jnp.float32),
        mesh=_sc_mesh(),
        scratch_types=[
            pltpu.VMEM((CH, EB), jnp.int32),
            pltpu.VMEM((CH, EB), jnp.int32),
            pltpu.VMEM((EB, 128), jnp.float32),
            pltpu.VMEM((EB, 128), jnp.float32),
            pltpu.VMEM((EB, 128), jnp.float32),
            pltpu.VMEM_SHARED((NPAD, 128), jnp.float32),
            pltpu.SemaphoreType.DMA,
            pltpu.SemaphoreType.DMA,
            pltpu.SemaphoreType.DMA,
        ],
    )
    return k(a, src2d, dst2d)


def _sc_degree(dst2d):
    k = pl.kernel(
        _degree_body,
        out_type=jax.ShapeDtypeStruct((2, NPAD, 128), jnp.float32),
        mesh=_sc_mesh(),
        scratch_types=[
            pltpu.VMEM((NB, EB), jnp.int32),
            pltpu.VMEM((EB, 128), jnp.float32),
            pltpu.VMEM((EB, 128), jnp.float32),
            pltpu.VMEM_SHARED((NPAD, 128), jnp.float32),
            pltpu.SemaphoreType.DMA,
        ],
    )
    return k(dst2d)


def _debug_kernel_sc_only(omics_features, edge_index, topo_features, w_o1, b_o1, g_o1, be_o1,
           w_o2, b_o2, g_o2, be_o2, w_t, b_t, g_t, be_t, w_g1, b_g1, g_l1,
           be_l1, w_g2, b_g2, g_l2, be_l2, w_c1, b_c1, g_c, be_c, w_c2, b_c2):
    src = edge_index[0]
    dst = edge_index[1]
    pad = EPAD - E_EDGES
    srcp = jnp.concatenate([src, jnp.zeros((pad,), src.dtype)])
    dstp = jnp.concatenate([dst, jnp.full((pad,), N_NODES, dst.dtype)])
    src2d = srcp.reshape(EPAD // EB, EB)
    dst2d = dstp.reshape(EPAD // EB, EB)

    degp = _sc_degree(dst2d)
    deg = degp[0, :N_NODES, 0] + degp[1, :N_NODES, 0] + 2.0
    dinv = lax.rsqrt(deg)

    h = omics_features @ w_o1 + b_o1
    h = jax.nn.relu(_ln(h, g_o1, be_o1))
    h = h @ w_o2 + b_o2
    h_om = _ln(h, g_o2, be_o2)
    ht = topo_features @ w_t + b_t
    h_tp = jax.nn.relu(_ln(ht, g_t, be_t))
    h0 = jnp.concatenate([h_om, h_tp], axis=-1)

    def gcn_epilogue(xw, s, b, g, be):
        c = dinv[:, None] * s + 2.0 * (dinv ** 2)[:, None] * xw + b
        hh = 0.8 * c + 0.2 * h0
        return _elu(_ln(hh, g, be))

    xw1 = h0 @ w_g1
    a1 = dinv[:, None] * xw1
    s1 = _sc_scatter(a1, src2d, dst2d)
    s = s1[0, :N_NODES, :] + s1[1, :N_NODES, :]
    h1 = gcn_epilogue(xw1, s, b_g1, g_l1, be_l1)

    xw2 = h1 @ w_g2
    a2 = dinv[:, None] * xw2
    s2 = _sc_scatter(a2, src2d, dst2d)
    s = s2[0, :N_NODES, :] + s2[1, :N_NODES, :]
    h2 = gcn_epilogue(xw2, s, b_g2, g_l2, be_l2)

    z = jax.nn.relu(_ln(h2 @ w_c1 + b_c1, g_c, be_c))
    return (z @ w_c2 + b_c2)[:, 0]


def _real_kernel(omics_features, edge_index, topo_features, w_o1, b_o1, g_o1, be_o1,
           w_o2, b_o2, g_o2, be_o2, w_t, b_t, g_t, be_t, w_g1, b_g1, g_l1,
           be_l1, w_g2, b_g2, g_l2, be_l2, w_c1, b_c1, g_c, be_c, w_c2, b_c2):
    f32 = jnp.float32
    r = lambda v: v.reshape(1, -1).astype(f32)

    src = edge_index[0]
    dst = edge_index[1]
    pad = EPAD - E_EDGES
    srcp = jnp.concatenate([src, jnp.zeros((pad,), src.dtype)])
    dstp = jnp.concatenate([dst, jnp.full((pad,), N_NODES, dst.dtype)])
    src2d = srcp.reshape(EPAD // EB, EB)
    dst2d = dstp.reshape(EPAD // EB, EB)

    degp = _sc_degree(dst2d)
    deg0 = degp[0, :N_NODES, :16]
    deg1 = degp[1, :N_NODES, :16]

    h0, xw1 = _tc_call(
        _enc_body, [omics_features, topo_features],
        [w_o1, r(b_o1), r(g_o1), r(be_o1), w_o2, r(b_o2), r(g_o2), r(be_o2),
         w_t, r(b_t), r(g_t), r(be_t), w_g1],
        [128, 128])

    a1 = _tc_call(_scale_body, [deg0, deg1, xw1], [], [128])

    s1 = _sc_scatter(a1, src2d, dst2d)
    s1a = s1[0, :N_NODES, :]
    s1b = s1[1, :N_NODES, :]

    xw2, a2 = _tc_call(
        _layer_body, [s1a, s1b, xw1, h0, deg0, deg1],
        [r(b_g1), r(g_l1), r(be_l1), w_g2],
        [128, 128])

    s2 = _sc_scatter(a2, src2d, dst2d)
    s2a = s2[0, :N_NODES, :]
    s2b = s2[1, :N_NODES, :]

    logits = _tc_call(
        _final_body, [s2a, s2b, xw2, h0, deg0, deg1],
        [r(b_g2), r(g_l2), r(be_l2), w_c1, r(b_c1), r(g_c), r(be_c),
         r(w_c2), r(b_c2)],
        [1])

    return logits[:, 0]


def _debug_kernel_deg_only(*args):
    import jax.numpy as _jnp
    edge_index = args[1]
    src = edge_index[0]; dst = edge_index[1]

    def scatter_jnp(a, src2d, dst2d):
        s = jax.ops.segment_sum(a[src], dst, num_segments=N_NODES)
        spad = _jnp.zeros((NPAD, 128), _jnp.float32).at[:N_NODES].set(s)
        return _jnp.stack([spad, _jnp.zeros_like(spad)])

    global _sc_scatter
    orig = _sc_scatter
    _sc_scatter = scatter_jnp
    try:
        return _debug_kernel_sc_only(*args)
    finally:
        _sc_scatter = orig


def _gather_body(a_hbm, src2d, out_hbm, src_v, buf0, sem0):
    cid = lax.axis_index("c")
    sid = lax.axis_index("s")
    wid = sid * 2 + cid
    pltpu.sync_copy(src2d.at[pl.ds(wid * NB, NB)], src_v)

    def body(t, c):
        pltpu.async_copy(a_hbm.at[src_v.at[t]], buf0, sem0).wait()
        pltpu.sync_copy(buf0, out_hbm.at[pl.ds((wid * NB + t) * EB, EB)])
        return c

    lax.fori_loop(0, NB, body, 0)


def _sc_gather_test(a, src2d):
    k = pl.kernel(
        _gather_body,
        out_type=jax.ShapeDtypeStruct((EPAD, 128), jnp.float32),
        mesh=_sc_mesh(),
        scratch_types=[
            pltpu.VMEM((NB, EB), jnp.int32),
            pltpu.VMEM((EB, 128), jnp.float32),
            pltpu.SemaphoreType.DMA,
        ],
    )
    return k(a, src2d)


def _debug_kernel_gather(*args):
    edge_index = args[1]
    src = edge_index[0]
    pad = EPAD - E_EDGES
    srcp = jnp.concatenate([src, jnp.zeros((pad,), src.dtype)])
    src2d = srcp.reshape(EPAD // EB, EB)
    a = jnp.arange(N_NODES, dtype=jnp.float32)[:, None] * jnp.ones((1, 128), jnp.float32)
    g = _sc_gather_test(a, src2d)
    ref = a[srcp]
    err = jnp.sum(jnp.abs(g - ref))
    out = _debug_kernel_deg_only(*args)
    return out + err


kernel = _real_kernel


# speedup vs baseline: 1.9629x; 1.9629x over previous
import functools

import jax
import jax.numpy as jnp
from jax import lax
from jax.experimental import pallas as pl
from jax.experimental.pallas import tpu as pltpu
from jax.experimental.pallas import tpu_sc as plsc

N_NODES = 10000
NPAD = 10112
E_EDGES = 320000
EB = 64
NW = 32
NB = 160
EPAD = NW * NB * EB
RPS = NPAD // 16
ROW_CHUNKS = [(i * EB, EB) for i in range(RPS // EB)] + [((RPS // EB) * EB, RPS % EB)]
RBLK = 1000
GRID = N_NODES // RBLK


def _ln(x, g, b, eps=1e-5):
    m = jnp.mean(x, axis=-1, keepdims=True)
    v = jnp.mean((x - m) ** 2, axis=-1, keepdims=True)
    return (x - m) * lax.rsqrt(v + eps) * g + b


def _elu(x):
    return jnp.where(x > 0, x, jnp.exp(x) - 1.0)


def _dinv(deg0_ref, deg1_ref):
    d = deg0_ref[:, 0:1] + deg1_ref[:, 0:1] + 2.0
    return lax.rsqrt(d)


def _enc_body(om_ref, tp_ref, wo1, bo1, go1, beo1, wo2, bo2, go2, beo2,
              wt, bt, gt, bet, wg1, h0_ref, xw1_ref):
    h = jnp.dot(om_ref[...], wo1[...], preferred_element_type=jnp.float32) + bo1[...]
    h = jax.nn.relu(_ln(h, go1[...], beo1[...]))
    h = jnp.dot(h, wo2[...], preferred_element_type=jnp.float32) + bo2[...]
    h_om = _ln(h, go2[...], beo2[...])
    ht = jnp.dot(tp_ref[...], wt[...], preferred_element_type=jnp.float32) + bt[...]
    h_tp = jax.nn.relu(_ln(ht, gt[...], bet[...]))
    h0 = jnp.concatenate([h_om, h_tp], axis=-1)
    h0_ref[...] = h0
    xw1_ref[...] = jnp.dot(h0, wg1[...], preferred_element_type=jnp.float32)


def _scale_body(deg0_ref, deg1_ref, xw_ref, a_ref):
    a_ref[...] = _dinv(deg0_ref, deg1_ref) * xw_ref[...]


def _layer_body(s0_ref, s1_ref, xw_ref, h0_ref, deg0_ref, deg1_ref,
                bg, gl, bel, wg2, xw2_ref, a2_ref):
    dinv = _dinv(deg0_ref, deg1_ref)
    c = dinv * (s0_ref[...] + s1_ref[...]) + (2.0 * dinv * dinv) * xw_ref[...] + bg[...]
    h = 0.8 * c + 0.2 * h0_ref[...]
    h = _elu(_ln(h, gl[...], bel[...]))
    xw2 = jnp.dot(h, wg2[...], preferred_element_type=jnp.float32)
    xw2_ref[...] = xw2
    a2_ref[...] = dinv * xw2


def _final_body(s0_ref, s1_ref, xw_ref, h0_ref, deg0_ref, deg1_ref,
                bg, gl, bel, wc1, bc1, gc, bec, wc2r, bc2, out_ref):
    dinv = _dinv(deg0_ref, deg1_ref)
    c = dinv * (s0_ref[...] + s1_ref[...]) + (2.0 * dinv * dinv) * xw_ref[...] + bg[...]
    h = 0.8 * c + 0.2 * h0_ref[...]
    h = _elu(_ln(h, gl[...], bel[...]))
    z = jnp.dot(h, wc1[...], preferred_element_type=jnp.float32) + bc1[...]
    z = jax.nn.relu(_ln(z, gc[...], bec[...]))
    out_ref[...] = jnp.sum(z * wc2r[...], axis=-1, keepdims=True) + bc2[...]


def _row_spec(cols):
    return pl.BlockSpec((RBLK, cols), lambda i: (i, 0))


def _full_spec(shape):
    nd = len(shape)
    return pl.BlockSpec(shape, lambda i: (0,) * nd)


def _tc_call(body, ins, full_ins, outs):
    in_specs = ([_row_spec(x.shape[1]) for x in ins]
                + [_full_spec(w.shape) for w in full_ins])
    out_specs = [_row_spec(c) for c in outs]
    out_shape = [jax.ShapeDtypeStruct((N_NODES, c), jnp.float32) for c in outs]
    res = pl.pallas_call(
        body, grid=(GRID,), in_specs=in_specs,
        out_specs=out_specs if len(outs) > 1 else out_specs[0],
        out_shape=out_shape if len(outs) > 1 else out_shape[0],
    )(*ins, *full_ins)
    return res


def _zero_rows(buf, ncols):
    nchunks = ncols // 16

    def zrow(i, c):
        for j in range(nchunks):
            buf[i, pl.ds(j * 16, 16)] = jnp.zeros((16,), jnp.float32)
        return c

    lax.fori_loop(0, buf.shape[0], zrow, 0)


def _fill_ones(buf, ncols):
    nchunks = ncols // 16

    def orow(i, c):
        for j in range(nchunks):
            buf[i, pl.ds(j * 16, 16)] = jnp.ones((16,), jnp.float32)
        return c

    lax.fori_loop(0, buf.shape[0], orow, 0)


CH = 32
NCHUNK = NB // CH


def _scatter_rows_body(a_hbm, src2d, dst2d, out_hbm,
                       src_v, dst_v, buf0, buf1, acc, sem0, sem1):
    cid = lax.axis_index("c")
    sid = lax.axis_index("s")
    wid = sid * 2 + cid
    _zero_rows(buf0, 128)
    base = sid * RPS
    for off, sz in ROW_CHUNKS:
        pltpu.sync_copy(buf0.at[pl.ds(0, sz)], acc.at[pl.ds(base + off, sz)])
    plsc.subcore_barrier()

    def fire(t, buf, sem):
        pltpu.async_copy(a_hbm.at[src_v.at[t]], buf, sem)

    def drain(buf, sem):
        pltpu.make_async_copy(a_hbm.at[pl.ds(0, EB)], buf, sem).wait()

    def scat(t, buf):
        pltpu.sync_copy(buf, acc.at[dst_v.at[t]], add=True)

    def chunk(c, carry):
        pltpu.sync_copy(src2d.at[pl.ds(wid * NB + c * CH, CH)], src_v)
        pltpu.sync_copy(dst2d.at[pl.ds(wid * NB + c * CH, CH)], dst_v)
        fire(0, buf0, sem0)

        def pair(k, cc):
            a = 2 * k
            fire(a + 1, buf1, sem1)
            drain(buf0, sem0)
            scat(a, buf0)

            @pl.when(k < CH // 2 - 1)
            def _():
                fire(a + 2, buf0, sem0)

            drain(buf1, sem1)
            scat(a + 1, buf1)
            return cc

        lax.fori_loop(0, CH // 2, pair, 0)
        return carry

    lax.fori_loop(0, NCHUNK, chunk, 0)
    plsc.subcore_barrier()
    for off, sz in ROW_CHUNKS:
        pltpu.sync_copy(acc.at[pl.ds(base + off, sz)], buf0.at[pl.ds(0, sz)])
        pltpu.sync_copy(buf0.at[pl.ds(0, sz)], out_hbm.at[cid, pl.ds(base + off, sz)])


def _degree_body(dst2d, out_hbm, dst_v, ones_v, zbuf, acc, _unused_sem):
    cid = lax.axis_index("c")
    sid = lax.axis_index("s")
    wid = sid * 2 + cid
    pltpu.sync_copy(dst2d.at[pl.ds(wid * NB, NB)], dst_v)
    _zero_rows(zbuf, 128)
    _fill_ones(ones_v, 128)
    base = sid * RPS
    for off, sz in ROW_CHUNKS:
        pltpu.sync_copy(zbuf.at[pl.ds(0, sz)], acc.at[pl.ds(base + off, sz)])
    plsc.subcore_barrier()

    def body(t, c):
        pltpu.sync_copy(ones_v, acc.at[dst_v.at[t]], add=True)
        return c

    lax.fori_loop(0, NB, body, 0)
    plsc.subcore_barrier()
    for off, sz in ROW_CHUNKS:
        pltpu.sync_copy(acc.at[pl.ds(base + off, sz)], zbuf.at[pl.ds(0, sz)])
        pltpu.sync_copy(zbuf.at[pl.ds(0, sz)], out_hbm.at[cid, pl.ds(base + off, sz)])


def _sc_mesh():
    return plsc.VectorSubcoreMesh(core_axis_name="c", subcore_axis_name="s",
                                  num_cores=2, num_subcores=16)


def _sc_scatter(a, src2d, dst2d):
    k = pl.kernel(
        _scatter_rows_body,
        out_type=jax.ShapeDtypeStruct((2, NPAD, 128), jnp.float32),
        mesh=_sc_mesh(),
        scratch_types=[
            pltpu.VMEM((CH, EB), jnp.int32),
            pltpu.VMEM((CH, EB), jnp.int32),
            pltpu.VMEM((EB, 128), jnp.float32),
            pltpu.VMEM((EB, 128), jnp.float32),
            pltpu.VMEM_SHARED((NPAD, 128), jnp.float32),
            pltpu.SemaphoreType.DMA,
            pltpu.SemaphoreType.DMA,
        ],
    )
    return k(a, src2d, dst2d)


def _sc_degree(dst2d):
    k = pl.kernel(
        _degree_body,
        out_type=jax.ShapeDtypeStruct((2, NPAD, 128), jnp.float32),
        mesh=_sc_mesh(),
        scratch_types=[
            pltpu.VMEM((NB, EB), jnp.int32),
            pltpu.VMEM((EB, 128), jnp.float32),
            pltpu.VMEM((EB, 128), jnp.float32),
            pltpu.VMEM_SHARED((NPAD, 128), jnp.float32),
            pltpu.SemaphoreType.DMA,
        ],
    )
    return k(dst2d)


def _real_kernel(omics_features, edge_index, topo_features, w_o1, b_o1, g_o1, be_o1,
           w_o2, b_o2, g_o2, be_o2, w_t, b_t, g_t, be_t, w_g1, b_g1, g_l1,
           be_l1, w_g2, b_g2, g_l2, be_l2, w_c1, b_c1, g_c, be_c, w_c2, b_c2):
    f32 = jnp.float32
    r = lambda v: v.reshape(1, -1).astype(f32)

    src = edge_index[0]
    dst = edge_index[1]
    pad = EPAD - E_EDGES
    srcp = jnp.concatenate([src, jnp.zeros((pad,), src.dtype)])
    dstp = jnp.concatenate([dst, jnp.full((pad,), N_NODES, dst.dtype)])
    src2d = srcp.reshape(EPAD // EB, EB)
    dst2d = dstp.reshape(EPAD // EB, EB)

    degp = _sc_degree(dst2d)
    deg0 = degp[0, :N_NODES, :16]
    deg1 = degp[1, :N_NODES, :16]

    h0, xw1 = _tc_call(
        _enc_body, [omics_features, topo_features],
        [w_o1, r(b_o1), r(g_o1), r(be_o1), w_o2, r(b_o2), r(g_o2), r(be_o2),
         w_t, r(b_t), r(g_t), r(be_t), w_g1],
        [128, 128])

    a1 = _tc_call(_scale_body, [deg0, deg1, xw1], [], [128])

    s1 = _sc_scatter(a1, src2d, dst2d)
    s1a = s1[0, :N_NODES, :]
    s1b = s1[1, :N_NODES, :]

    xw2, a2 = _tc_call(
        _layer_body, [s1a, s1b, xw1, h0, deg0, deg1],
        [r(b_g1), r(g_l1), r(be_l1), w_g2],
        [128, 128])

    s2 = _sc_scatter(a2, src2d, dst2d)
    s2a = s2[0, :N_NODES, :]
    s2b = s2[1, :N_NODES, :]

    logits = _tc_call(
        _final_body, [s2a, s2b, xw2, h0, deg0, deg1],
        [r(b_g2), r(g_l2), r(be_l2), w_c1, r(b_c1), r(g_c), r(be_c),
         r(w_c2), r(b_c2)],
        [1])

    return logits[:, 0]


kernel = _real_kernel


# speedup vs baseline: 1.9777x; 1.0075x over previous
import functools

import jax
import jax.numpy as jnp
from jax import lax
from jax.experimental import pallas as pl
from jax.experimental.pallas import tpu as pltpu
from jax.experimental.pallas import tpu_sc as plsc

N_NODES = 10000
NPAD = 10112
E_EDGES = 320000
EB = 64
NW = 32
NB = 160
EPAD = NW * NB * EB
RPS = NPAD // 16
ROW_CHUNKS = [(i * EB, EB) for i in range(RPS // EB)] + [((RPS // EB) * EB, RPS % EB)]
RBLK = 1000
GRID = N_NODES // RBLK


def _ln(x, g, b, eps=1e-5):
    m = jnp.mean(x, axis=-1, keepdims=True)
    v = jnp.mean((x - m) ** 2, axis=-1, keepdims=True)
    return (x - m) * lax.rsqrt(v + eps) * g + b


def _elu(x):
    return jnp.where(x > 0, x, jnp.exp(x) - 1.0)


def _dinv(deg0_ref, deg1_ref):
    d = deg0_ref[:, 0:1] + deg1_ref[:, 0:1] + 2.0
    return lax.rsqrt(d)


def _enc_body(om_ref, tp_ref, wo1, bo1, go1, beo1, wo2, bo2, go2, beo2,
              wt, bt, gt, bet, wg1, h0_ref, xw1_ref):
    h = jnp.dot(om_ref[...], wo1[...], preferred_element_type=jnp.float32) + bo1[...]
    h = jax.nn.relu(_ln(h, go1[...], beo1[...]))
    h = jnp.dot(h, wo2[...], preferred_element_type=jnp.float32) + bo2[...]
    h_om = _ln(h, go2[...], beo2[...])
    ht = jnp.dot(tp_ref[...], wt[...], preferred_element_type=jnp.float32) + bt[...]
    h_tp = jax.nn.relu(_ln(ht, gt[...], bet[...]))
    h0 = jnp.concatenate([h_om, h_tp], axis=-1)
    h0_ref[...] = h0
    xw1_ref[...] = jnp.dot(h0, wg1[...], preferred_element_type=jnp.float32)


def _scale_body(deg0_ref, deg1_ref, xw_ref, a_ref):
    a_ref[...] = _dinv(deg0_ref, deg1_ref) * xw_ref[...]


def _layer_body(s0_ref, s1_ref, xw_ref, h0_ref, deg0_ref, deg1_ref,
                bg, gl, bel, wg2, xw2_ref, a2_ref):
    dinv = _dinv(deg0_ref, deg1_ref)
    c = dinv * (s0_ref[...] + s1_ref[...]) + (2.0 * dinv * dinv) * xw_ref[...] + bg[...]
    h = 0.8 * c + 0.2 * h0_ref[...]
    h = _elu(_ln(h, gl[...], bel[...]))
    xw2 = jnp.dot(h, wg2[...], preferred_element_type=jnp.float32)
    xw2_ref[...] = xw2
    a2_ref[...] = dinv * xw2


def _final_body(s0_ref, s1_ref, xw_ref, h0_ref, deg0_ref, deg1_ref,
                bg, gl, bel, wc1, bc1, gc, bec, wc2r, bc2, out_ref):
    dinv = _dinv(deg0_ref, deg1_ref)
    c = dinv * (s0_ref[...] + s1_ref[...]) + (2.0 * dinv * dinv) * xw_ref[...] + bg[...]
    h = 0.8 * c + 0.2 * h0_ref[...]
    h = _elu(_ln(h, gl[...], bel[...]))
    z = jnp.dot(h, wc1[...], preferred_element_type=jnp.float32) + bc1[...]
    z = jax.nn.relu(_ln(z, gc[...], bec[...]))
    out_ref[...] = jnp.sum(z * wc2r[...], axis=-1, keepdims=True) + bc2[...]


def _row_spec(cols):
    return pl.BlockSpec((RBLK, cols), lambda i: (i, 0))


def _full_spec(shape):
    nd = len(shape)
    return pl.BlockSpec(shape, lambda i: (0,) * nd)


def _tc_call(body, ins, full_ins, outs):
    in_specs = ([_row_spec(x.shape[1]) for x in ins]
                + [_full_spec(w.shape) for w in full_ins])
    out_specs = [_row_spec(c) for c in outs]
    out_shape = [jax.ShapeDtypeStruct((N_NODES, c), jnp.float32) for c in outs]
    res = pl.pallas_call(
        body, grid=(GRID,), in_specs=in_specs,
        out_specs=out_specs if len(outs) > 1 else out_specs[0],
        out_shape=out_shape if len(outs) > 1 else out_shape[0],
    )(*ins, *full_ins)
    return res


def _zero_rows(buf, ncols):
    nchunks = ncols // 16

    def zrow(i, c):
        for j in range(nchunks):
            buf[i, pl.ds(j * 16, 16)] = jnp.zeros((16,), jnp.float32)
        return c

    lax.fori_loop(0, buf.shape[0], zrow, 0)


def _fill_ones(buf, ncols):
    nchunks = ncols // 16

    def orow(i, c):
        for j in range(nchunks):
            buf[i, pl.ds(j * 16, 16)] = jnp.ones((16,), jnp.float32)
        return c

    lax.fori_loop(0, buf.shape[0], orow, 0)


CH = 80
NCHUNK = NB // CH


def _scatter_rows_body(a_hbm, src2d, dst2d, out_hbm,
                       src_v, dst_v, buf0, buf1, acc, sem0, sem1):
    cid = lax.axis_index("c")
    sid = lax.axis_index("s")
    wid = sid * 2 + cid
    _zero_rows(buf0, 128)
    base = sid * RPS
    for off, sz in ROW_CHUNKS:
        pltpu.sync_copy(buf0.at[pl.ds(0, sz)], acc.at[pl.ds(base + off, sz)])
    plsc.subcore_barrier()

    def fire(t, buf, sem):
        pltpu.async_copy(a_hbm.at[src_v.at[t]], buf, sem)

    def drain(buf, sem):
        pltpu.make_async_copy(a_hbm.at[pl.ds(0, EB)], buf, sem).wait()

    def scat(t, buf):
        pltpu.sync_copy(buf, acc.at[dst_v.at[t]], add=True)

    def chunk(c, carry):
        pltpu.sync_copy(src2d.at[pl.ds(wid * NB + c * CH, CH)], src_v)
        pltpu.sync_copy(dst2d.at[pl.ds(wid * NB + c * CH, CH)], dst_v)
        fire(0, buf0, sem0)

        def pair(k, cc):
            a = 2 * k
            fire(a + 1, buf1, sem1)
            drain(buf0, sem0)
            scat(a, buf0)

            @pl.when(k < CH // 2 - 1)
            def _():
                fire(a + 2, buf0, sem0)

            drain(buf1, sem1)
            scat(a + 1, buf1)
            return cc

        lax.fori_loop(0, CH // 2, pair, 0)
        return carry

    lax.fori_loop(0, NCHUNK, chunk, 0)
    plsc.subcore_barrier()
    for off, sz in ROW_CHUNKS:
        pltpu.sync_copy(acc.at[pl.ds(base + off, sz)], buf0.at[pl.ds(0, sz)])
        pltpu.sync_copy(buf0.at[pl.ds(0, sz)], out_hbm.at[cid, pl.ds(base + off, sz)])


def _degree_body(dst2d, out_hbm, dst_v, ones_v, zbuf, acc, _unused_sem):
    cid = lax.axis_index("c")
    sid = lax.axis_index("s")
    wid = sid * 2 + cid
    pltpu.sync_copy(dst2d.at[pl.ds(wid * NB, NB)], dst_v)
    _zero_rows(zbuf, 128)
    _fill_ones(ones_v, 128)
    base = sid * RPS
    for off, sz in ROW_CHUNKS:
        pltpu.sync_copy(zbuf.at[pl.ds(0, sz)], acc.at[pl.ds(base + off, sz)])
    plsc.subcore_barrier()

    def body(t, c):
        pltpu.sync_copy(ones_v, acc.at[dst_v.at[t]], add=True)
        return c

    lax.fori_loop(0, NB, body, 0)
    plsc.subcore_barrier()
    for off, sz in ROW_CHUNKS:
        pltpu.sync_copy(acc.at[pl.ds(base + off, sz)], zbuf.at[pl.ds(0, sz)])
        pltpu.sync_copy(zbuf.at[pl.ds(0, sz)], out_hbm.at[cid, pl.ds(base + off, sz)])


def _sc_mesh():
    return plsc.VectorSubcoreMesh(core_axis_name="c", subcore_axis_name="s",
                                  num_cores=2, num_subcores=16)


def _sc_scatter(a, src2d, dst2d):
    k = pl.kernel(
        _scatter_rows_body,
        out_type=jax.ShapeDtypeStruct((2, NPAD, 128), jnp.float32),
        mesh=_sc_mesh(),
        scratch_types=[
            pltpu.VMEM((CH, EB), jnp.int32),
            pltpu.VMEM((CH, EB), jnp.int32),
            pltpu.VMEM((EB, 128), jnp.float32),
            pltpu.VMEM((EB, 128), jnp.float32),
            pltpu.VMEM_SHARED((NPAD, 128), jnp.float32),
            pltpu.SemaphoreType.DMA,
            pltpu.SemaphoreType.DMA,
        ],
    )
    return k(a, src2d, dst2d)


def _sc_degree(dst2d):
    k = pl.kernel(
        _degree_body,
        out_type=jax.ShapeDtypeStruct((2, NPAD, 128), jnp.float32),
        mesh=_sc_mesh(),
        scratch_types=[
            pltpu.VMEM((NB, EB), jnp.int32),
            pltpu.VMEM((EB, 128), jnp.float32),
            pltpu.VMEM((EB, 128), jnp.float32),
            pltpu.VMEM_SHARED((NPAD, 128), jnp.float32),
            pltpu.SemaphoreType.DMA,
        ],
    )
    return k(dst2d)


def _real_kernel(omics_features, edge_index, topo_features, w_o1, b_o1, g_o1, be_o1,
           w_o2, b_o2, g_o2, be_o2, w_t, b_t, g_t, be_t, w_g1, b_g1, g_l1,
           be_l1, w_g2, b_g2, g_l2, be_l2, w_c1, b_c1, g_c, be_c, w_c2, b_c2):
    f32 = jnp.float32
    r = lambda v: v.reshape(1, -1).astype(f32)

    src = edge_index[0]
    dst = edge_index[1]
    pad = EPAD - E_EDGES
    srcp = jnp.concatenate([src, jnp.zeros((pad,), src.dtype)])
    dstp = jnp.concatenate([dst, jnp.full((pad,), N_NODES, dst.dtype)])
    src2d = srcp.reshape(EPAD // EB, EB)
    dst2d = dstp.reshape(EPAD // EB, EB)

    degp = _sc_degree(dst2d)
    deg0 = degp[0, :N_NODES, :16]
    deg1 = degp[1, :N_NODES, :16]

    h0, xw1 = _tc_call(
        _enc_body, [omics_features, topo_features],
        [w_o1, r(b_o1), r(g_o1), r(be_o1), w_o2, r(b_o2), r(g_o2), r(be_o2),
         w_t, r(b_t), r(g_t), r(be_t), w_g1],
        [128, 128])

    a1 = _tc_call(_scale_body, [deg0, deg1, xw1], [], [128])

    s1 = _sc_scatter(a1, src2d, dst2d)
    s1a = s1[0, :N_NODES, :]
    s1b = s1[1, :N_NODES, :]

    xw2, a2 = _tc_call(
        _layer_body, [s1a, s1b, xw1, h0, deg0, deg1],
        [r(b_g1), r(g_l1), r(be_l1), w_g2],
        [128, 128])

    s2 = _sc_scatter(a2, src2d, dst2d)
    s2a = s2[0, :N_NODES, :]
    s2b = s2[1, :N_NODES, :]

    logits = _tc_call(
        _final_body, [s2a, s2b, xw2, h0, deg0, deg1],
        [r(b_g2), r(g_l2), r(be_l2), w_c1, r(b_c1), r(g_c), r(be_c),
         r(w_c2), r(b_c2)],
        [1])

    return logits[:, 0]


kernel = _real_kernel
